# baseline (speedup 1.0000x reference)
import jax
import jax.numpy as jnp
import numpy as np
from jax import lax
from jax.experimental import pallas as pl
from jax.experimental.pallas import tpu as pltpu
from jax.experimental.pallas import tpu_sc as plsc

N = 10000
D = 128
R = 200
E = 320000

NC = 2
NS = 16
LANES = 16
NW = NC * NS

CHUNK = 64
NCHUNK = 162
EPT = NCHUNK * CHUNK
E_PAD = NW * EPT
assert E_PAD >= E and NCHUNK % 6 == 0

ACC_ROWS = 10240
ZPT = ACC_ROWS // NS
DUMMY_DST = N

_PERM = np.concatenate(
    [32 * b + np.concatenate([np.arange(0, 32, 2), np.arange(1, 32, 2)])
     for b in range(D // 32)])


def _sc_segment_sum(x_hbm, rel_hbm, idx_hbm, zeros_hbm, out_hbm,
                    acc, rel_sp, xb0, xb1, pb0, pb1, pb2, ix0, ix1, ix2,
                    dd0, dd1,
                    sem_x0, sem_x1, sem_r0, sem_r1, sem_r2,
                    sem_s0, sem_s1, sem_s2, sem_i0, sem_i1, sem_i2):
    c = lax.axis_index("c")
    s = lax.axis_index("s")
    tile = c * NS + s
    xb = (xb0, xb1)
    pb = (pb0, pb1, pb2)
    ix = (ix0, ix1, ix2)
    dd = (dd0, dd1)
    sem_x = (sem_x0, sem_x1)
    sem_r = (sem_r0, sem_r1, sem_r2)
    sem_s = (sem_s0, sem_s1, sem_s2)
    sem_i = (sem_i0, sem_i1, sem_i2)

    pltpu.sync_copy(zeros_hbm, acc.at[pl.ds(s * ZPT, ZPT)])
    @pl.when(s == 0)
    def _():
        pltpu.sync_copy(rel_hbm, rel_sp)
    plsc.subcore_barrier()

    def issue_idx(k, rs):
        pltpu.async_copy(idx_hbm.at[tile, k], ix[rs % 3], sem_i[rs % 3])

    def wait_idx(rs):
        pltpu.make_async_copy(idx_hbm.at[tile, 0], ix[rs % 3],
                              sem_i[rs % 3]).wait()

    def issue_gather(rs):
        b3 = rs % 3
        pltpu.async_copy(x_hbm.at[ix[b3].at[0]], xb[rs % 2], sem_x[rs % 2])
        pltpu.async_copy(rel_sp.at[ix[b3].at[2]], pb[b3], sem_r[b3])

    def wait_gather(rs):
        b3 = rs % 3
        pltpu.make_async_copy(x_hbm.at[ix[b3].at[0]], xb[rs % 2],
                              sem_x[rs % 2]).wait()
        pltpu.make_async_copy(rel_sp.at[ix[b3].at[2]], pb[b3],
                              sem_r[b3]).wait()

    def issue_scatter(rs):
        pltpu.async_copy(pb[rs % 3], acc.at[dd[rs % 2]], sem_s[rs % 3],
                         add=True)

    def wait_scatter(rs):
        pltpu.make_async_copy(pb[rs % 3], acc.at[dd[rs % 2]],
                              sem_s[rs % 3]).wait()

    def step(k, rs, head=False, tail=False, last_idx=False):
        b2, b3 = rs % 2, rs % 3
        wait_gather(rs)
        for j in range(CHUNK // LANES):
            sl = pl.ds(j * LANES, LANES)
            dd[b2][sl] = ix[b3][1, sl]

        @plsc.parallel_loop(0, CHUNK, unroll=4)
        def _(i):
            for j in range(D // 32):
                xw = xb[b2][i, pl.ds(j * LANES, LANES)]
                ev = lax.bitcast_convert_type(xw << 16, jnp.float32)
                od = lax.bitcast_convert_type(xw & jnp.int32(-65536),
                                              jnp.float32)
                sle = pl.ds(j * 32, LANES)
                slo = pl.ds(j * 32 + LANES, LANES)
                pb[b3][i, sle] = pb[b3][i, sle] * ev
                pb[b3][i, slo] = pb[b3][i, slo] * od
        issue_scatter(rs)
        if not last_idx:
            issue_idx(k + 3, rs + 3)
        if not head:
            wait_scatter(rs - 1)
        if not tail:
            wait_idx(rs + 2)
            issue_gather(rs + 2)

    issue_idx(0, 0)
    issue_idx(1, 1)
    issue_idx(2, 2)
    wait_idx(0)
    issue_gather(0)
    wait_idx(1)
    issue_gather(1)
    for k in range(6):
        step(k, k, head=(k == 0))

    def body(g, carry):
        for r in range(6):
            step(g * 6 + r, r)
        return carry

    lax.fori_loop(1, NCHUNK // 6 - 1, body, 0)

    for k in range(NCHUNK - 6, NCHUNK):
        step(k, k, tail=(k >= NCHUNK - 2), last_idx=(k + 3 >= NCHUNK))
    wait_scatter(NCHUNK - 1)

    plsc.subcore_barrier()
    pltpu.sync_copy(acc.at[pl.ds(s * ZPT, ZPT)],
                    out_hbm.at[c, pl.ds(s * ZPT, ZPT)])


def _tc_finish_body(s_ref, x_ref, norm_ref, w_ref, lw_ref, o_ref):
    agg = s_ref[0] + s_ref[1]
    o_ref[...] = (
        jnp.dot(agg, w_ref[...], preferred_element_type=jnp.float32)
        * norm_ref[...]
        + jnp.dot(x_ref[...], lw_ref[...], preferred_element_type=jnp.float32)
    )


def kernel(x, norm, prev_h, emb_rel, edge_index, edge_type,
           weight_neighbor, loop_weight):
    del prev_h
    src = edge_index[0]
    dst = edge_index[1]
    pad = E_PAD - E
    ar = jnp.arange(pad, dtype=jnp.int32)
    src_p = jnp.concatenate([src, ar % N])
    dst_p = jnp.concatenate([dst, DUMMY_DST + ar % (ACC_ROWS - N)])
    typ_p = jnp.concatenate([edge_type, ar % R])
    idx_all = jnp.stack(
        [src_p.reshape(NW, NCHUNK, CHUNK),
         dst_p.reshape(NW, NCHUNK, CHUNK),
         typ_p.reshape(NW, NCHUNK, CHUNK)], axis=2)
    zeros_blk = jnp.zeros((ZPT, D), jnp.float32)
    perm = jnp.asarray(_PERM)
    x_bf = jax.lax.bitcast_convert_type(
        x.astype(jnp.bfloat16).reshape(N, D // 2, 2), jnp.int32)
    rel_perm = emb_rel[:, perm]
    w_perm = weight_neighbor[perm, :]

    mesh = plsc.VectorSubcoreMesh(core_axis_name="c", subcore_axis_name="s",
                                  num_cores=NC, num_subcores=NS)
    partial = pl.kernel(
        _sc_segment_sum,
        out_type=jax.ShapeDtypeStruct((NC, ACC_ROWS, D), jnp.float32),
        mesh=mesh,
        compiler_params=pltpu.CompilerParams(use_tc_tiling_on_sc=False),
        scratch_types=[
            pltpu.VMEM_SHARED((ACC_ROWS, D), jnp.float32),
            pltpu.VMEM_SHARED((R, D), jnp.float32),
            pltpu.VMEM((CHUNK, D // 2), jnp.int32),
            pltpu.VMEM((CHUNK, D // 2), jnp.int32),
            pltpu.VMEM((CHUNK, D), jnp.float32),
            pltpu.VMEM((CHUNK, D), jnp.float32),
            pltpu.VMEM((CHUNK, D), jnp.float32),
            pltpu.VMEM((3, CHUNK), jnp.int32),
            pltpu.VMEM((3, CHUNK), jnp.int32),
            pltpu.VMEM((3, CHUNK), jnp.int32),
            pltpu.VMEM((CHUNK,), jnp.int32),
            pltpu.VMEM((CHUNK,), jnp.int32),
        ] + [pltpu.SemaphoreType.DMA] * 11,
    )(x_bf, rel_perm, idx_all, zeros_blk)

    blk = 1000
    out = pl.pallas_call(
        _tc_finish_body,
        grid=(N // blk,),
        in_specs=[
            pl.BlockSpec((NC, blk, D), lambda i: (0, i, 0)),
            pl.BlockSpec((blk, D), lambda i: (i, 0)),
            pl.BlockSpec((blk, 1), lambda i: (i, 0)),
            pl.BlockSpec((D, D), lambda i: (0, 0)),
            pl.BlockSpec((D, D), lambda i: (0, 0)),
        ],
        out_specs=pl.BlockSpec((blk, D), lambda i: (i, 0)),
        out_shape=jax.ShapeDtypeStruct((N, D), jnp.float32),
    )(partial, x, norm, w_perm, loop_weight)
    return out


# speedup vs baseline: 1.0769x; 1.0769x over previous
import jax
import jax.numpy as jnp
from jax import lax
from jax.experimental import pallas as pl
from jax.experimental.pallas import tpu as pltpu
from jax.experimental.pallas import tpu_sc as plsc

N = 10000
D = 128
R = 200
E = 320000

NC = 2
NS = 16
LANES = 16
NW = NC * NS

CHUNK = 64
NCHUNK = 162
EPT = NCHUNK * CHUNK
E_PAD = NW * EPT
assert E_PAD >= E and NCHUNK % 6 == 0

ACC_ROWS = 10240
ZPT = ACC_ROWS // NS
DUMMY_DST = N


def _sc_segment_sum(x_hbm, rel_hbm, idx_hbm, zeros_hbm, out_hbm,
                    acc, rel_sp, xb0, xb1, pb0, pb1, pb2, ix0, ix1, ix2,
                    dd0, dd1,
                    sem_x0, sem_x1, sem_r0, sem_r1, sem_r2,
                    sem_s0, sem_s1, sem_s2, sem_i0, sem_i1, sem_i2):
    c = lax.axis_index("c")
    s = lax.axis_index("s")
    tile = c * NS + s
    xb = (xb0, xb1)
    pb = (pb0, pb1, pb2)
    ix = (ix0, ix1, ix2)
    dd = (dd0, dd1)
    sem_x = (sem_x0, sem_x1)
    sem_r = (sem_r0, sem_r1, sem_r2)
    sem_s = (sem_s0, sem_s1, sem_s2)
    sem_i = (sem_i0, sem_i1, sem_i2)

    pltpu.sync_copy(zeros_hbm, acc.at[pl.ds(s * ZPT, ZPT)])
    @pl.when(s == 0)
    def _():
        pltpu.sync_copy(rel_hbm, rel_sp)
    plsc.subcore_barrier()

    def issue_idx(k, rs):
        pltpu.async_copy(idx_hbm.at[tile, k], ix[rs % 3], sem_i[rs % 3])

    def wait_idx(rs):
        pltpu.make_async_copy(idx_hbm.at[tile, 0], ix[rs % 3],
                              sem_i[rs % 3]).wait()

    def issue_gather(rs):
        b3 = rs % 3
        pltpu.async_copy(x_hbm.at[ix[b3].at[0]], xb[rs % 2], sem_x[rs % 2])
        pltpu.async_copy(rel_sp.at[ix[b3].at[2]], pb[b3], sem_r[b3])

    def wait_gather(rs):
        b3 = rs % 3
        pltpu.make_async_copy(x_hbm.at[ix[b3].at[0]], xb[rs % 2],
                              sem_x[rs % 2]).wait()
        pltpu.make_async_copy(rel_sp.at[ix[b3].at[2]], pb[b3],
                              sem_r[b3]).wait()

    def issue_scatter(rs):
        pltpu.async_copy(pb[rs % 3], acc.at[dd[rs % 2]], sem_s[rs % 3],
                         add=True)

    def wait_scatter(rs):
        pltpu.make_async_copy(pb[rs % 3], acc.at[dd[rs % 2]],
                              sem_s[rs % 3]).wait()

    def step(k, rs, head=False, tail=False, last_idx=False):
        b2, b3 = rs % 2, rs % 3
        wait_gather(rs)
        for j in range(CHUNK // LANES):
            sl = pl.ds(j * LANES, LANES)
            dd[b2][sl] = ix[b3][1, sl]

        @plsc.parallel_loop(0, CHUNK, unroll=4)
        def _(i):
            for j in range(D // LANES):
                sl = pl.ds(j * LANES, LANES)
                pb[b3][i, sl] = pb[b3][i, sl] * xb[b2][i, sl]
        issue_scatter(rs)
        if not last_idx:
            issue_idx(k + 3, rs + 3)
        if not head:
            wait_scatter(rs - 1)
        if not tail:
            wait_idx(rs + 2)
            issue_gather(rs + 2)

    issue_idx(0, 0)
    issue_idx(1, 1)
    issue_idx(2, 2)
    wait_idx(0)
    issue_gather(0)
    wait_idx(1)
    issue_gather(1)
    for k in range(6):
        step(k, k, head=(k == 0))

    def body(g, carry):
        for r in range(6):
            step(g * 6 + r, r)
        return carry

    lax.fori_loop(1, NCHUNK // 6 - 1, body, 0)

    for k in range(NCHUNK - 6, NCHUNK):
        step(k, k, tail=(k >= NCHUNK - 2), last_idx=(k + 3 >= NCHUNK))
    wait_scatter(NCHUNK - 1)

    plsc.subcore_barrier()
    pltpu.sync_copy(acc.at[pl.ds(s * ZPT, ZPT)],
                    out_hbm.at[c, pl.ds(s * ZPT, ZPT)])


def _tc_finish_body(s_ref, x_ref, norm_ref, w_ref, lw_ref, o_ref):
    agg = s_ref[0] + s_ref[1]
    o_ref[...] = (
        jnp.dot(agg, w_ref[...], preferred_element_type=jnp.float32)
        * norm_ref[...]
        + jnp.dot(x_ref[...], lw_ref[...], preferred_element_type=jnp.float32)
    )


def kernel(x, norm, prev_h, emb_rel, edge_index, edge_type,
           weight_neighbor, loop_weight):
    del prev_h
    src = edge_index[0]
    dst = edge_index[1]
    pad = E_PAD - E
    ar = jnp.arange(pad, dtype=jnp.int32)
    src_p = jnp.concatenate([src, ar % N])
    dst_p = jnp.concatenate([dst, DUMMY_DST + ar % (ACC_ROWS - N)])
    typ_p = jnp.concatenate([edge_type, ar % R])
    idx_all = jnp.stack(
        [src_p.reshape(NW, NCHUNK, CHUNK),
         dst_p.reshape(NW, NCHUNK, CHUNK),
         typ_p.reshape(NW, NCHUNK, CHUNK)], axis=2)
    zeros_blk = jnp.zeros((ZPT, D), jnp.float32)

    mesh = plsc.VectorSubcoreMesh(core_axis_name="c", subcore_axis_name="s",
                                  num_cores=NC, num_subcores=NS)
    partial = pl.kernel(
        _sc_segment_sum,
        out_type=jax.ShapeDtypeStruct((NC, ACC_ROWS, D), jnp.float32),
        mesh=mesh,
        scratch_types=[
            pltpu.VMEM_SHARED((ACC_ROWS, D), jnp.float32),
            pltpu.VMEM_SHARED((R, D), jnp.float32),
            pltpu.VMEM((CHUNK, D), jnp.float32),
            pltpu.VMEM((CHUNK, D), jnp.float32),
            pltpu.VMEM((CHUNK, D), jnp.float32),
            pltpu.VMEM((CHUNK, D), jnp.float32),
            pltpu.VMEM((CHUNK, D), jnp.float32),
            pltpu.VMEM((3, CHUNK), jnp.int32),
            pltpu.VMEM((3, CHUNK), jnp.int32),
            pltpu.VMEM((3, CHUNK), jnp.int32),
            pltpu.VMEM((CHUNK,), jnp.int32),
            pltpu.VMEM((CHUNK,), jnp.int32),
        ] + [pltpu.SemaphoreType.DMA] * 11,
    )(x, emb_rel, idx_all, zeros_blk)

    blk = 1000
    out = pl.pallas_call(
        _tc_finish_body,
        grid=(N // blk,),
        in_specs=[
            pl.BlockSpec((NC, blk, D), lambda i: (0, i, 0)),
            pl.BlockSpec((blk, D), lambda i: (i, 0)),
            pl.BlockSpec((blk, 1), lambda i: (i, 0)),
            pl.BlockSpec((D, D), lambda i: (0, 0)),
            pl.BlockSpec((D, D), lambda i: (0, 0)),
        ],
        out_specs=pl.BlockSpec((blk, D), lambda i: (i, 0)),
        out_shape=jax.ShapeDtypeStruct((N, D), jnp.float32),
    )(partial, x, norm, weight_neighbor, loop_weight)
    return out
